# baseline (speedup 1.0000x reference)
import functools

import jax
import jax.numpy as jnp
from jax import lax
from jax.experimental import pallas as pl
from jax.experimental.pallas import tpu as pltpu
from jax.experimental.pallas import tpu_sc as plsc

NUM_EMBEDDINGS = 100000
EMBEDDING_DIM = 128
BATCH = 4096
FIELDS = 26

_NC = 2
_NS = 16
_NW = _NC * _NS

_C = 128
_NCHUNK = BATCH * FIELDS // (_NW * _C)
_CHUNKS_PER_F = BATCH // _C

_NBUF = 7
_LOOKAHEAD = 5


def _make_sc_gather():
  mesh = plsc.VectorSubcoreMesh(core_axis_name="c", subcore_axis_name="s")

  bufs = [pltpu.VMEM((_C, EMBEDDING_DIM), jnp.float32) for _ in range(_NBUF)]
  gsems = [pltpu.SemaphoreType.DMA for _ in range(_NBUF)]
  ssems = [pltpu.SemaphoreType.DMA for _ in range(_NBUF)]
  isems = [pltpu.SemaphoreType.DMA for _ in range(_NBUF)]

  @functools.partial(
      pl.kernel,
      mesh=mesh,
      out_type=jax.ShapeDtypeStruct((FIELDS, BATCH, EMBEDDING_DIM),
                                    jnp.float32),
      compiler_params=pltpu.CompilerParams(use_tc_tiling_on_sc=True),
      scratch_types=[pltpu.VMEM((_NCHUNK, _C), jnp.int32)]
      + bufs + gsems + ssems + isems,
  )
  def sc_gather(idx_hbm, table_hbm, out_hbm, idx_v, *scratch):
    buf = scratch[:_NBUF]
    gsem = scratch[_NBUF:2 * _NBUF]
    ssem = scratch[2 * _NBUF:3 * _NBUF]
    isem = scratch[3 * _NBUF:]
    wid = lax.axis_index("s") * _NC + lax.axis_index("c")
    c0 = wid * _NCHUNK

    def chunk_coords(k):
      c = c0 + k
      return c // _CHUNKS_PER_F, (c % _CHUNKS_PER_F) * _C

    idxcps = {}
    gathers = {}
    scatters = {}

    def start_idx(k):
      f, b0 = chunk_coords(k)
      return pltpu.async_copy(
          idx_hbm.at[f, pl.ds(b0, _C)], idx_v.at[k], isem[k % _NBUF])

    def start_gather(k):
      return pltpu.async_copy(
          table_hbm.at[idx_v.at[k]], buf[k % _NBUF], gsem[k % _NBUF])

    def start_scatter(k):
      f, b0 = chunk_coords(k)
      return pltpu.async_copy(
          buf[k % _NBUF], out_hbm.at[f, pl.ds(b0, _C)], ssem[k % _NBUF])

    for k in range(-_LOOKAHEAD - 1, _NCHUNK):
      ki = k + _LOOKAHEAD + 1
      if 0 <= ki < _NCHUNK:
        idxcps[ki] = start_idx(ki)
      kg = k + _LOOKAHEAD
      if 0 <= kg < _NCHUNK:
        prev = kg - _NBUF
        if prev >= 0:
          scatters[prev].wait()
        idxcps[kg].wait()
        gathers[kg] = start_gather(kg)
      if k >= 0:
        gathers[k].wait()
        scatters[k] = start_scatter(k)
    for k in range(max(0, _NCHUNK - _NBUF), _NCHUNK):
      scatters[k].wait()

  return sc_gather


_sc_gather = _make_sc_gather()


@jax.jit
def kernel(input, weight):
  idx_t = input.astype(jnp.int32).T
  out_fmajor = _sc_gather(idx_t, weight)
  return out_fmajor.transpose(1, 0, 2)


# speedup vs baseline: 1.0278x; 1.0278x over previous
import functools

import jax
import jax.numpy as jnp
from jax import lax
from jax.experimental import pallas as pl
from jax.experimental.pallas import tpu as pltpu
from jax.experimental.pallas import tpu_sc as plsc

NUM_EMBEDDINGS = 100000
EMBEDDING_DIM = 128
BATCH = 4096
FIELDS = 26

_NC = 2
_NS = 16
_NW = _NC * _NS

_C = 128
_NCHUNK = BATCH * FIELDS // (_NW * _C)
_CHUNKS_PER_F = BATCH // _C

_NBUF = 3
_LOOKAHEAD = 3


def _make_sc_gather():
  mesh = plsc.VectorSubcoreMesh(core_axis_name="c", subcore_axis_name="s")

  bufs = [pltpu.VMEM((2 * _C, EMBEDDING_DIM), jnp.float32) for _ in range(_NBUF)]
  gsems = [pltpu.SemaphoreType.DMA for _ in range(_NBUF)]
  ssems = [pltpu.SemaphoreType.DMA for _ in range(_NBUF)]

  @functools.partial(
      pl.kernel,
      mesh=mesh,
      out_type=jax.ShapeDtypeStruct((FIELDS, BATCH, EMBEDDING_DIM),
                                    jnp.float32),
      compiler_params=pltpu.CompilerParams(use_tc_tiling_on_sc=True),
      scratch_types=[pltpu.VMEM((_NCHUNK * _C,), jnp.int32)] + bufs + gsems + ssems,
  )
  def sc_gather(idx_hbm, table_hbm, out_hbm, idx_v, *scratch):
    buf = scratch[:_NBUF]
    gsem = scratch[_NBUF:2 * _NBUF]
    ssem = scratch[2 * _NBUF:]
    wid = lax.axis_index("s") * _NC + lax.axis_index("c")
    c0 = wid * _NCHUNK
    pltpu.sync_copy(idx_hbm.at[wid], idx_v)

    gathers = {}
    scatters = {}

    def start_gather(k):
      g, half = divmod(k, 2)
      return pltpu.async_copy(
          table_hbm.at[idx_v.at[pl.ds(k * _C, _C)]],
          buf[g % _NBUF].at[pl.ds(half * _C, _C)], gsem[g % _NBUF])

    def start_scatter(g):
      c = c0 + g * 2
      f = c // _CHUNKS_PER_F
      b0 = (c % _CHUNKS_PER_F) * _C
      return pltpu.async_copy(
          buf[g % _NBUF], out_hbm.at[f, pl.ds(b0, 2 * _C)], ssem[g % _NBUF])

    for k in range(-_LOOKAHEAD, _NCHUNK):
      nk = k + _LOOKAHEAD
      if nk < _NCHUNK:
        prev = nk // 2 - _NBUF
        if prev >= 0 and nk % 2 == 0:
          scatters[prev].wait()
        gathers[nk] = start_gather(nk)
      if k >= 0 and k % 2 == 1:
        gathers[k - 1].wait()
        gathers[k].wait()
        scatters[k // 2] = start_scatter(k // 2)
    for g in range(max(0, _NCHUNK // 2 - _NBUF), _NCHUNK // 2):
      scatters[g].wait()

  return sc_gather


_sc_gather = _make_sc_gather()


@jax.jit
def kernel(input, weight):
  idx = input.astype(jnp.int32).T.reshape(_NW, _NCHUNK * _C)
  out_fmajor = _sc_gather(idx, weight)
  return out_fmajor.transpose(1, 0, 2)


# speedup vs baseline: 1.0353x; 1.0073x over previous
import functools

import jax
import jax.numpy as jnp
from jax import lax
from jax.experimental import pallas as pl
from jax.experimental.pallas import tpu as pltpu
from jax.experimental.pallas import tpu_sc as plsc

NUM_EMBEDDINGS = 100000
EMBEDDING_DIM = 128
BATCH = 4096
FIELDS = 26

_NC = 2
_NS = 16
_NW = _NC * _NS

_C = 128
_NCHUNK = BATCH * FIELDS // (_NW * _C)
_CHUNKS_PER_F = BATCH // _C

_NBUF = 7
_LOOKAHEAD = 5


def _make_sc_gather():
  mesh = plsc.VectorSubcoreMesh(core_axis_name="c", subcore_axis_name="s")

  bufs = [pltpu.VMEM((_C, EMBEDDING_DIM), jnp.float32) for _ in range(_NBUF)]
  gsems = [pltpu.SemaphoreType.DMA for _ in range(_NBUF)]
  ssems = [pltpu.SemaphoreType.DMA for _ in range(_NBUF)]

  @functools.partial(
      pl.kernel,
      mesh=mesh,
      out_type=jax.ShapeDtypeStruct((FIELDS, BATCH, EMBEDDING_DIM),
                                    jnp.float32),
      compiler_params=pltpu.CompilerParams(use_tc_tiling_on_sc=True),
      scratch_types=[pltpu.VMEM((2, BATCH), jnp.int32)] + bufs + gsems + ssems,
  )
  def sc_gather(idx_hbm, table_hbm, out_hbm, idx_win, *scratch):
    buf = scratch[:_NBUF]
    gsem = scratch[_NBUF:2 * _NBUF]
    ssem = scratch[2 * _NBUF:]
    wid = lax.axis_index("s") * _NC + lax.axis_index("c")
    c0 = wid * _NCHUNK
    f0 = jnp.minimum(c0 // _CHUNKS_PER_F, FIELDS - 2)
    pltpu.sync_copy(idx_hbm.at[f0], idx_win.at[0])
    pltpu.sync_copy(idx_hbm.at[f0 + 1], idx_win.at[1])

    gathers = {}
    scatters = {}

    def chunk_coords(k):
      c = c0 + k
      return c // _CHUNKS_PER_F, (c % _CHUNKS_PER_F) * _C

    def start_gather(k):
      f, b0 = chunk_coords(k)
      return pltpu.async_copy(
          table_hbm.at[idx_win.at[f - f0, pl.ds(b0, _C)]], buf[k % _NBUF],
          gsem[k % _NBUF])

    def start_scatter(k):
      f, b0 = chunk_coords(k)
      return pltpu.async_copy(
          buf[k % _NBUF], out_hbm.at[f, pl.ds(b0, _C)], ssem[k % _NBUF])

    for k in range(-_LOOKAHEAD, _NCHUNK):
      nk = k + _LOOKAHEAD
      if nk < _NCHUNK:
        prev = nk - _NBUF
        if prev >= 0:
          scatters[prev].wait()
        gathers[nk] = start_gather(nk)
      if k >= 0:
        gathers[k].wait()
        scatters[k] = start_scatter(k)
    for k in range(max(0, _NCHUNK - _NBUF), _NCHUNK):
      scatters[k].wait()

  return sc_gather


_sc_gather = _make_sc_gather()


@jax.jit
def kernel(input, weight):
  idx_t = input.astype(jnp.int32).T
  out_fmajor = _sc_gather(idx_t, weight)
  return out_fmajor.transpose(1, 0, 2)
